# Initial kernel scaffold; baseline (speedup 1.0000x reference)
import jax
import jax.numpy as jnp
from jax.experimental import pallas as pl


def kernel(x, edge_index, edge_weight, W1, b1, W2, b2, Wm1, bm1, Wm2, bm2):
    raise NotImplementedError("write your pallas kernel here")


# speedup vs baseline: 5.7820x; 5.7820x over previous
import functools

import jax
import jax.numpy as jnp
from jax import lax
from jax.experimental import pallas as pl
from jax.experimental.pallas import tpu as pltpu
from jax.experimental.pallas import tpu_sc as plsc

_N = 10000
_E = 160000
_NP = 10240
_G = 80
_EP = 163840
_EF = _EP + _NP
_ER = _EP // _G
_EFR = _EF // _G
_RPT = _ER // 16
_FRPT = _EFR // 16
_NT = _NP // 16


def _rsqrt_nr(v):
    i = lax.bitcast_convert_type(v, jnp.int32)
    i = jnp.int32(0x5F3759DF) - lax.shift_right_arithmetic(i, jnp.int32(1))
    y = lax.bitcast_convert_type(i, jnp.float32)
    for _ in range(3):
        y = y * (1.5 - 0.5 * v * y * y)
    return y


def _prep_body(row2d, col2d, ew2d, norm_out,
               row_v, col_v, ew_v, norm_v, dinv_v, slice_v, zbuf,
               deg_sh, dinv_sh):
    c = lax.axis_index("c")
    s = lax.axis_index("s")

    def _z(i, _):
        zbuf[pl.ds(i * 16, 16)] = jnp.zeros((16,), jnp.float32)
        return _
    lax.fori_loop(0, _NP // 16, _z, None)

    @pl.when(s == 0)
    def _():
        pltpu.sync_copy(zbuf, deg_sh)
    plsc.subcore_barrier()

    r0 = pl.multiple_of(s * _RPT, 1)
    pltpu.sync_copy(row2d.at[pl.ds(r0, _RPT)], row_v)
    pltpu.sync_copy(col2d.at[pl.ds(r0, _RPT)], col_v)
    pltpu.sync_copy(ew2d.at[pl.ds(r0, _RPT)], ew_v)

    def _deg(j, _):
        pltpu.sync_copy(ew_v.at[j], deg_sh.at[col_v.at[j]], add=True)
        return _
    lax.fori_loop(0, _RPT, _deg, None)
    plsc.subcore_barrier()

    n0 = pl.multiple_of(s * _NT, 8)
    pltpu.sync_copy(deg_sh.at[pl.ds(n0, _NT)], slice_v)

    def _dv(i, _):
        d = slice_v[pl.ds(i * 16, 16)] + 1.0
        slice_v[pl.ds(i * 16, 16)] = _rsqrt_nr(d)
        return _
    lax.fori_loop(0, _NT // 16, _dv, None)
    pltpu.sync_copy(slice_v, dinv_sh.at[pl.ds(n0, _NT)])
    plsc.subcore_barrier()

    pltpu.sync_copy(dinv_sh, dinv_v)

    @pl.when(c == 0)
    def _():
        def _nm(j, _):
            for q in range(_G // 16):
                sl = pl.ds(q * 16, 16)
                rv = row_v[j, sl]
                cv = col_v[j, sl]
                wv = ew_v[j, sl]
                dr = plsc.load_gather(dinv_v, [rv])
                dc = plsc.load_gather(dinv_v, [cv])
                norm_v[j, sl] = dr * wv * dc
            return _
        lax.fori_loop(0, _RPT, _nm, None)
        pltpu.sync_copy(norm_v, norm_out.at[pl.ds(r0, _RPT)])

    @pl.when(c == 1)
    def _():
        for rr in range(_NT // _G):
            for q in range(_G // 16):
                dv = slice_v[pl.ds(rr * _G + q * 16, 16)]
                norm_v[rr, pl.ds(q * 16, 16)] = dv * dv
        pltpu.sync_copy(norm_v.at[pl.ds(0, _NT // _G)],
                        norm_out.at[pl.ds(_ER + s * (_NT // _G), _NT // _G)])


def _sc_prep(row2d, col2d, ew2d):
    mesh = plsc.VectorSubcoreMesh(core_axis_name="c", subcore_axis_name="s")
    f = functools.partial(
        pl.kernel, _prep_body, mesh=mesh,
        compiler_params=pltpu.CompilerParams(needs_layout_passes=False),
        out_type=jax.ShapeDtypeStruct((_EFR, _G), jnp.float32),
        scratch_types=[
            pltpu.VMEM((_RPT, _G), jnp.int32),
            pltpu.VMEM((_RPT, _G), jnp.int32),
            pltpu.VMEM((_RPT, _G), jnp.float32),
            pltpu.VMEM((_RPT, _G), jnp.float32),
            pltpu.VMEM((_NP,), jnp.float32),
            pltpu.VMEM((_NT,), jnp.float32),
            pltpu.VMEM((_NP,), jnp.float32),
            pltpu.VMEM_SHARED((_NP,), jnp.float32),
            pltpu.VMEM_SHARED((_NP,), jnp.float32),
        ],
    )
    return f()(row2d, col2d, ew2d)


def _conv_body(hview, row2d, col2d, norm2d, out,
               row_v, col_v, norm_v, gidx, rows_v, sem, acc_sh):
    c = lax.axis_index("c")
    s = lax.axis_index("s")

    def _z(i, _):
        for q in range(8):
            rows_v[i, pl.ds(q * 16, 16)] = jnp.zeros((16,), jnp.float32)
        return _
    lax.fori_loop(0, _G, _z, None)
    n0 = pl.multiple_of(s * _NT, 8)
    for m in range(_NT // _G):
        pltpu.sync_copy(rows_v, acc_sh.at[pl.ds(n0 + m * _G, _G)])
    plsc.subcore_barrier()

    r0 = pl.multiple_of(s * _FRPT, 8)

    def _block(b, _):
        rb = pl.multiple_of(r0 + b * 8, 8)
        pltpu.sync_copy(row2d.at[pl.ds(rb, 8)], row_v)
        pltpu.sync_copy(col2d.at[pl.ds(rb, 8)], col_v)
        pltpu.sync_copy(norm2d.at[pl.ds(rb, 8)], norm_v)

        def _chunk(k, _):
            for q in range(_G // 16):
                sl = pl.ds(q * 16, 16)
                gidx[sl] = row_v[k, sl] * 2 + c
            pltpu.async_copy(hview.at[gidx], rows_v, sem).wait()

            def _sc(j, _):
                sp = plsc.load_gather(
                    norm_v, [jnp.full((16,), k, jnp.int32),
                             jnp.full((16,), j, jnp.int32)])
                for q in range(8):
                    sl = pl.ds(q * 16, 16)
                    rows_v[j, sl] = rows_v[j, sl] * sp
                return _
            lax.fori_loop(0, _G, _sc, None)

            pltpu.sync_copy(rows_v, acc_sh.at[col_v.at[k]], add=True)
            return _
        lax.fori_loop(0, 8, _chunk, None)
        return _
    lax.fori_loop(0, _FRPT // 8, _block, None)
    plsc.subcore_barrier()

    pltpu.sync_copy(acc_sh.at[pl.ds(n0, _NT)],
                    out.at[pl.ds(n0, _NT), pl.ds(c * 128, 128)])


def _sc_conv(hview, row2d, col2d, norm2d):
    mesh = plsc.VectorSubcoreMesh(core_axis_name="c", subcore_axis_name="s")
    f = functools.partial(
        pl.kernel, _conv_body, mesh=mesh,
        compiler_params=pltpu.CompilerParams(needs_layout_passes=False),
        out_type=jax.ShapeDtypeStruct((_NP, 256), jnp.float32),
        scratch_types=[
            pltpu.VMEM((8, _G), jnp.int32),
            pltpu.VMEM((8, _G), jnp.int32),
            pltpu.VMEM((8, _G), jnp.float32),
            pltpu.VMEM((_G,), jnp.int32),
            pltpu.VMEM((_G, 128), jnp.float32),
            pltpu.SemaphoreType.DMA,
            pltpu.VMEM_SHARED((_NP, 128), jnp.float32),
        ],
    )
    return f()(hview, row2d, col2d, norm2d)


_BM = 1000


def _mm_body(x_ref, w_ref, o_ref):
    o_ref[...] = jnp.dot(x_ref[...], w_ref[...],
                         preferred_element_type=jnp.float32)


def _tc_matmul(x, w):
    n, kdim = x.shape
    m = w.shape[1]
    return pl.pallas_call(
        _mm_body,
        grid=(n // _BM,),
        in_specs=[
            pl.BlockSpec((_BM, kdim), lambda i: (i, 0)),
            pl.BlockSpec((kdim, m), lambda i: (0, 0)),
        ],
        out_specs=pl.BlockSpec((_BM, m), lambda i: (i, 0)),
        out_shape=jax.ShapeDtypeStruct((n, m), jnp.float32),
    )(x, w)


def _fuse_body(msg_ref, b_ref, w_ref, o_ref):
    t = jnp.maximum(msg_ref[...] + b_ref[...], 0.0)
    o_ref[...] = jnp.dot(t, w_ref[...], preferred_element_type=jnp.float32)


def _tc_fuse(msg, b, w):
    return pl.pallas_call(
        _fuse_body,
        grid=(_N // _BM,),
        in_specs=[
            pl.BlockSpec((_BM, 256), lambda i: (i, 0)),
            pl.BlockSpec((1, 256), lambda i: (0, 0)),
            pl.BlockSpec((256, 256), lambda i: (0, 0)),
        ],
        out_specs=pl.BlockSpec((_BM, 256), lambda i: (i, 0)),
        out_shape=jax.ShapeDtypeStruct((_N, 256), jnp.float32),
    )(msg, b.reshape(1, 256), w)


def _final_body(msg_ref, b_ref, wm1_ref, bm1_ref, wm2_ref, bm2_ref,
                ss_ref, cs_ref):
    i = pl.program_id(0)
    t = jnp.maximum(msg_ref[...] + b_ref[...], 0.0)
    z = jnp.dot(t, wm1_ref[...], preferred_element_type=jnp.float32)
    z = z + bm1_ref[...]
    z = jnp.dot(z, wm2_ref[...], preferred_element_type=jnp.float32)
    z = z + bm2_ref[...]
    z = z - jnp.max(z, axis=-1, keepdims=True)
    ez = jnp.exp(z)
    sm = ez / jnp.sum(ez, axis=-1, keepdims=True)

    @pl.when(i == 0)
    def _():
        ss_ref[...] = jnp.zeros_like(ss_ref)
        cs_ref[...] = jnp.zeros_like(cs_ref)

    ss_ref[...] += lax.dot_general(
        sm, sm, (((0,), (0,)), ((), ())), preferred_element_type=jnp.float32)
    cs_ref[...] += jnp.sum(sm, axis=0, keepdims=True)


def _tc_final(msg, b, wm1, bm1, wm2, bm2):
    return pl.pallas_call(
        _final_body,
        grid=(_N // _BM,),
        in_specs=[
            pl.BlockSpec((_BM, 256), lambda i: (i, 0)),
            pl.BlockSpec((1, 256), lambda i: (0, 0)),
            pl.BlockSpec((256, 256), lambda i: (0, 0)),
            pl.BlockSpec((1, 256), lambda i: (0, 0)),
            pl.BlockSpec((256, 16), lambda i: (0, 0)),
            pl.BlockSpec((1, 16), lambda i: (0, 0)),
        ],
        out_specs=[
            pl.BlockSpec((16, 16), lambda i: (0, 0)),
            pl.BlockSpec((1, 16), lambda i: (0, 0)),
        ],
        out_shape=[
            jax.ShapeDtypeStruct((16, 16), jnp.float32),
            jax.ShapeDtypeStruct((1, 16), jnp.float32),
        ],
    )(msg, b.reshape(1, 256), wm1, bm1.reshape(1, 256), wm2,
      bm2.reshape(1, 16))


def kernel(x, edge_index, edge_weight, W1, b1, W2, b2, Wm1, bm1, Wm2, bm2):
    epad = jnp.zeros((_EP - _E,), jnp.int32)
    row = jnp.concatenate([edge_index[0], epad])
    col = jnp.concatenate([edge_index[1], epad])
    ew = jnp.concatenate([edge_weight, epad.astype(jnp.float32)])

    row2d = row.reshape(_ER, _G)
    col2d = col.reshape(_ER, _G)
    ew2d = ew.reshape(_ER, _G)

    self_row = jnp.concatenate(
        [jnp.arange(_N, dtype=jnp.int32),
         jnp.zeros((_NP - _N,), jnp.int32)])
    self_col = jnp.arange(_NP, dtype=jnp.int32)
    row_f = jnp.concatenate([row, self_row]).reshape(_EFR, _G)
    col_f = jnp.concatenate([col, self_col]).reshape(_EFR, _G)

    norm2d = _sc_prep(row2d, col2d, ew2d)

    h1 = _tc_matmul(x, W1)
    msg1 = _sc_conv(h1.reshape(2 * _N, 128), row_f, col_f, norm2d)
    h2 = _tc_fuse(msg1, b1, W2)
    msg2 = _sc_conv(h2.reshape(2 * _N, 128), row_f, col_f, norm2d)
    ss, cs = _tc_final(msg2, b2, Wm1, bm1, Wm2, bm2)

    ssn = ss / jnp.sqrt(jnp.sum(ss * ss))
    ortho = jnp.sqrt(jnp.sum((ssn - jnp.eye(16, dtype=jnp.float32) / 4.0) ** 2))
    cluster = jnp.sqrt(jnp.sum(cs * cs)) / jnp.float32(_N) * 4.0 - 1.0
    return ortho + cluster


# speedup vs baseline: 7.4179x; 1.2829x over previous
import functools

import jax
import jax.numpy as jnp
from jax import lax
from jax.experimental import pallas as pl
from jax.experimental.pallas import tpu as pltpu
from jax.experimental.pallas import tpu_sc as plsc

_N = 10000
_E = 160000
_NP = 10240
_G = 80
_EP = 163840
_EF = _EP + _NP
_ER = _EP // _G
_EFR = _EF // _G
_RPT = _ER // 16
_FRPT = _EFR // 16
_NT = _NP // 16


def _rsqrt_nr(v):
    i = lax.bitcast_convert_type(v, jnp.int32)
    i = jnp.int32(0x5F3759DF) - lax.shift_right_arithmetic(i, jnp.int32(1))
    y = lax.bitcast_convert_type(i, jnp.float32)
    for _ in range(3):
        y = y * (1.5 - 0.5 * v * y * y)
    return y


def _prep_body(row2d, col2d, ew2d, norm_out,
               row_v, col_v, ew_v, norm_v, dinv_v, slice_v, zbuf,
               deg_sh, dinv_sh):
    c = lax.axis_index("c")
    s = lax.axis_index("s")

    def _z(i, _):
        zbuf[pl.ds(i * 16, 16)] = jnp.zeros((16,), jnp.float32)
        return _
    lax.fori_loop(0, _NP // 16, _z, None)

    @pl.when(s == 0)
    def _():
        pltpu.sync_copy(zbuf, deg_sh)
    plsc.subcore_barrier()

    r0 = pl.multiple_of(s * _RPT, 1)
    pltpu.sync_copy(row2d.at[pl.ds(r0, _RPT)], row_v)
    pltpu.sync_copy(col2d.at[pl.ds(r0, _RPT)], col_v)
    pltpu.sync_copy(ew2d.at[pl.ds(r0, _RPT)], ew_v)

    def _deg(j, _):
        pltpu.sync_copy(ew_v.at[j], deg_sh.at[col_v.at[j]], add=True)
        return _
    lax.fori_loop(0, _RPT, _deg, None)
    plsc.subcore_barrier()

    n0 = pl.multiple_of(s * _NT, 8)
    pltpu.sync_copy(deg_sh.at[pl.ds(n0, _NT)], slice_v)

    def _dv(i, _):
        d = slice_v[pl.ds(i * 16, 16)] + 1.0
        slice_v[pl.ds(i * 16, 16)] = _rsqrt_nr(d)
        return _
    lax.fori_loop(0, _NT // 16, _dv, None)
    pltpu.sync_copy(slice_v, dinv_sh.at[pl.ds(n0, _NT)])
    plsc.subcore_barrier()

    pltpu.sync_copy(dinv_sh, dinv_v)

    @pl.when(c == 0)
    def _():
        def _nm(j, _):
            for q in range(_G // 16):
                sl = pl.ds(q * 16, 16)
                rv = row_v[j, sl]
                cv = col_v[j, sl]
                wv = ew_v[j, sl]
                dr = plsc.load_gather(dinv_v, [rv])
                dc = plsc.load_gather(dinv_v, [cv])
                norm_v[j, sl] = dr * wv * dc
            return _
        lax.fori_loop(0, _RPT, _nm, None)
        pltpu.sync_copy(norm_v, norm_out.at[pl.ds(r0, _RPT)])

    @pl.when(c == 1)
    def _():
        for rr in range(_NT // _G):
            for q in range(_G // 16):
                dv = slice_v[pl.ds(rr * _G + q * 16, 16)]
                norm_v[rr, pl.ds(q * 16, 16)] = dv * dv
        pltpu.sync_copy(norm_v.at[pl.ds(0, _NT // _G)],
                        norm_out.at[pl.ds(_ER + s * (_NT // _G), _NT // _G)])


def _sc_prep(row2d, col2d, ew2d):
    mesh = plsc.VectorSubcoreMesh(core_axis_name="c", subcore_axis_name="s")
    f = functools.partial(
        pl.kernel, _prep_body, mesh=mesh,
        compiler_params=pltpu.CompilerParams(needs_layout_passes=False),
        out_type=jax.ShapeDtypeStruct((_EFR, _G), jnp.float32),
        scratch_types=[
            pltpu.VMEM((_RPT, _G), jnp.int32),
            pltpu.VMEM((_RPT, _G), jnp.int32),
            pltpu.VMEM((_RPT, _G), jnp.float32),
            pltpu.VMEM((_RPT, _G), jnp.float32),
            pltpu.VMEM((_NP,), jnp.float32),
            pltpu.VMEM((_NT,), jnp.float32),
            pltpu.VMEM((_NP,), jnp.float32),
            pltpu.VMEM_SHARED((_NP,), jnp.float32),
            pltpu.VMEM_SHARED((_NP,), jnp.float32),
        ],
    )
    return f()(row2d, col2d, ew2d)


def _conv_body(hview, row2d, col2d, norm2d, out,
               row_v, col_v, norm_v, gidx_a, gidx_b, rows_a, rows_b,
               sem_a, sem_b, acc_sh):
    c = lax.axis_index("c")
    s = lax.axis_index("s")

    def _z(i, _):
        for q in range(8):
            rows_a[i, pl.ds(q * 16, 16)] = jnp.zeros((16,), jnp.float32)
        return _
    lax.fori_loop(0, _G, _z, None)
    n0 = pl.multiple_of(s * _NT, 8)
    for m in range(_NT // _G):
        pltpu.sync_copy(rows_a, acc_sh.at[pl.ds(n0 + m * _G, _G)])
    plsc.subcore_barrier()

    r0 = pl.multiple_of(s * _FRPT, 8)
    bufs = [(gidx_a, rows_a, sem_a), (gidx_b, rows_b, sem_b)]

    def _gidx(k, g):
        for q in range(_G // 16):
            sl = pl.ds(q * 16, 16)
            g[sl] = row_v[k, sl] * 2 + c

    def _scale(k, r):
        def _sc(j, _):
            sp = plsc.load_gather(
                norm_v, [jnp.full((16,), k, jnp.int32),
                         jnp.full((16,), j, jnp.int32)])
            for q in range(8):
                sl = pl.ds(q * 16, 16)
                r[j, sl] = r[j, sl] * sp
            return _
        lax.fori_loop(0, _G, _sc, None)

    def _block(b, _):
        rb = pl.multiple_of(r0 + b * 8, 8)
        pltpu.sync_copy(row2d.at[pl.ds(rb, 8)], row_v)
        pltpu.sync_copy(col2d.at[pl.ds(rb, 8)], col_v)
        pltpu.sync_copy(norm2d.at[pl.ds(rb, 8)], norm_v)

        _gidx(0, bufs[0][0])
        cp = pltpu.async_copy(hview.at[bufs[0][0]], bufs[0][1], bufs[0][2])
        for k in range(8):
            g, r, se = bufs[k % 2]
            cp.wait()
            if k < 7:
                ng, nr, nse = bufs[(k + 1) % 2]
                _gidx(k + 1, ng)
                cp = pltpu.async_copy(hview.at[ng], nr, nse)
            _scale(k, r)
            pltpu.sync_copy(r, acc_sh.at[col_v.at[k]], add=True)
        return _
    lax.fori_loop(0, _FRPT // 8, _block, None)
    plsc.subcore_barrier()

    pltpu.sync_copy(acc_sh.at[pl.ds(n0, _NT)],
                    out.at[pl.ds(n0, _NT), pl.ds(c * 128, 128)])


def _sc_conv(hview, row2d, col2d, norm2d):
    mesh = plsc.VectorSubcoreMesh(core_axis_name="c", subcore_axis_name="s")
    f = functools.partial(
        pl.kernel, _conv_body, mesh=mesh,
        compiler_params=pltpu.CompilerParams(needs_layout_passes=False),
        out_type=jax.ShapeDtypeStruct((_NP, 256), jnp.float32),
        scratch_types=[
            pltpu.VMEM((8, _G), jnp.int32),
            pltpu.VMEM((8, _G), jnp.int32),
            pltpu.VMEM((8, _G), jnp.float32),
            pltpu.VMEM((_G,), jnp.int32),
            pltpu.VMEM((_G,), jnp.int32),
            pltpu.VMEM((_G, 128), jnp.float32),
            pltpu.VMEM((_G, 128), jnp.float32),
            pltpu.SemaphoreType.DMA,
            pltpu.SemaphoreType.DMA,
            pltpu.VMEM_SHARED((_NP, 128), jnp.float32),
        ],
    )
    return f()(hview, row2d, col2d, norm2d)


_BM = 1000


def _mm_body(x_ref, w_ref, o_ref):
    o_ref[...] = jnp.dot(x_ref[...], w_ref[...],
                         preferred_element_type=jnp.float32)


def _tc_matmul(x, w):
    n, kdim = x.shape
    m = w.shape[1]
    return pl.pallas_call(
        _mm_body,
        grid=(n // _BM,),
        in_specs=[
            pl.BlockSpec((_BM, kdim), lambda i: (i, 0)),
            pl.BlockSpec((kdim, m), lambda i: (0, 0)),
        ],
        out_specs=pl.BlockSpec((_BM, m), lambda i: (i, 0)),
        out_shape=jax.ShapeDtypeStruct((n, m), jnp.float32),
    )(x, w)


def _fuse_body(msg_ref, b_ref, w_ref, o_ref):
    t = jnp.maximum(msg_ref[...] + b_ref[...], 0.0)
    o_ref[...] = jnp.dot(t, w_ref[...], preferred_element_type=jnp.float32)


def _tc_fuse(msg, b, w):
    return pl.pallas_call(
        _fuse_body,
        grid=(_N // _BM,),
        in_specs=[
            pl.BlockSpec((_BM, 256), lambda i: (i, 0)),
            pl.BlockSpec((1, 256), lambda i: (0, 0)),
            pl.BlockSpec((256, 256), lambda i: (0, 0)),
        ],
        out_specs=pl.BlockSpec((_BM, 256), lambda i: (i, 0)),
        out_shape=jax.ShapeDtypeStruct((_N, 256), jnp.float32),
    )(msg, b.reshape(1, 256), w)


def _final_body(msg_ref, b_ref, wm1_ref, bm1_ref, wm2_ref, bm2_ref,
                ss_ref, cs_ref):
    i = pl.program_id(0)
    t = jnp.maximum(msg_ref[...] + b_ref[...], 0.0)
    z = jnp.dot(t, wm1_ref[...], preferred_element_type=jnp.float32)
    z = z + bm1_ref[...]
    z = jnp.dot(z, wm2_ref[...], preferred_element_type=jnp.float32)
    z = z + bm2_ref[...]
    z = z - jnp.max(z, axis=-1, keepdims=True)
    ez = jnp.exp(z)
    sm = ez / jnp.sum(ez, axis=-1, keepdims=True)

    @pl.when(i == 0)
    def _():
        ss_ref[...] = jnp.zeros_like(ss_ref)
        cs_ref[...] = jnp.zeros_like(cs_ref)

    ss_ref[...] += lax.dot_general(
        sm, sm, (((0,), (0,)), ((), ())), preferred_element_type=jnp.float32)
    cs_ref[...] += jnp.sum(sm, axis=0, keepdims=True)


def _tc_final(msg, b, wm1, bm1, wm2, bm2):
    return pl.pallas_call(
        _final_body,
        grid=(_N // _BM,),
        in_specs=[
            pl.BlockSpec((_BM, 256), lambda i: (i, 0)),
            pl.BlockSpec((1, 256), lambda i: (0, 0)),
            pl.BlockSpec((256, 256), lambda i: (0, 0)),
            pl.BlockSpec((1, 256), lambda i: (0, 0)),
            pl.BlockSpec((256, 16), lambda i: (0, 0)),
            pl.BlockSpec((1, 16), lambda i: (0, 0)),
        ],
        out_specs=[
            pl.BlockSpec((16, 16), lambda i: (0, 0)),
            pl.BlockSpec((1, 16), lambda i: (0, 0)),
        ],
        out_shape=[
            jax.ShapeDtypeStruct((16, 16), jnp.float32),
            jax.ShapeDtypeStruct((1, 16), jnp.float32),
        ],
    )(msg, b.reshape(1, 256), wm1, bm1.reshape(1, 256), wm2,
      bm2.reshape(1, 16))


def kernel(x, edge_index, edge_weight, W1, b1, W2, b2, Wm1, bm1, Wm2, bm2):
    epad = jnp.zeros((_EP - _E,), jnp.int32)
    row = jnp.concatenate([edge_index[0], epad])
    col = jnp.concatenate([edge_index[1], epad])
    ew = jnp.concatenate([edge_weight, epad.astype(jnp.float32)])

    row2d = row.reshape(_ER, _G)
    col2d = col.reshape(_ER, _G)
    ew2d = ew.reshape(_ER, _G)

    self_row = jnp.concatenate(
        [jnp.arange(_N, dtype=jnp.int32),
         jnp.zeros((_NP - _N,), jnp.int32)])
    self_col = jnp.arange(_NP, dtype=jnp.int32)
    row_f = jnp.concatenate([row, self_row]).reshape(_EFR, _G)
    col_f = jnp.concatenate([col, self_col]).reshape(_EFR, _G)

    norm2d = _sc_prep(row2d, col2d, ew2d)

    h1 = _tc_matmul(x, W1)
    msg1 = _sc_conv(h1.reshape(2 * _N, 128), row_f, col_f, norm2d)
    h2 = _tc_fuse(msg1, b1, W2)
    msg2 = _sc_conv(h2.reshape(2 * _N, 128), row_f, col_f, norm2d)
    ss, cs = _tc_final(msg2, b2, Wm1, bm1, Wm2, bm2)

    ssn = ss / jnp.sqrt(jnp.sum(ss * ss))
    ortho = jnp.sqrt(jnp.sum((ssn - jnp.eye(16, dtype=jnp.float32) / 4.0) ** 2))
    cluster = jnp.sqrt(jnp.sum(cs * cs)) / jnp.float32(_N) * 4.0 - 1.0
    return ortho + cluster
